# baseline (device time: 68366 ns/iter reference)
import jax
import jax.numpy as jnp
from jax import lax
from jax.experimental import pallas as pl
from jax.experimental.pallas import tpu as pltpu

N_DEV = 4
B_PER = 2
HQ_PER = 8
SQ = 512
SKV = 512
DH = 64
DMODEL = 768
HDIM = HQ_PER * DH
KVW = 32 * DH
BLK = 64
N_PHASE = N_DEV * B_PER

_CompilerParams = getattr(pltpu, "CompilerParams", None) or getattr(
    pltpu, "TPUCompilerParams"
)


def kernel(x, Wq, K_ext, V_ext, Wo):
    bf16 = jnp.bfloat16
    my = lax.axis_index("i")
    x_bf = x.astype(bf16)

    def prep(t):
        return lax.dynamic_slice_in_dim(
            t.reshape(8, SKV, KVW), my * B_PER, B_PER, axis=0
        ).astype(bf16)

    K_ord = prep(K_ext)
    V_ord = prep(V_ext)
    Wq_bf = Wq.astype(bf16)
    Wo_bf = Wo.astype(bf16)

    def body(
        x_ref, k_ref, v_ref, wq_ref, wo_ref, out_ref,
        comm_wq, comm_wo, q_ref, ctx_ref, bias_ref, kblk, vblk, vaug_ref,
        send_sems, recv_sems, blk_sems,
    ):
        my_i = lax.axis_index("i")
        left = lax.rem(my_i + N_DEV - 1, N_DEV)
        right = lax.rem(my_i + 1, N_DEV)
        origins = [my_i, left, right, lax.rem(my_i + 2, N_DEV)]

        def issue_blk(phase, buf):
            slot, b = divmod(phase, B_PER)
            off = origins[slot] * HDIM
            dmas = []
            for src, dst in ((k_ref, kblk), (v_ref, vblk)):
                d = pltpu.make_async_copy(
                    src.at[b, :, pl.ds(off, HDIM)],
                    dst.at[buf],
                    blk_sems.at[buf],
                )
                d.start()
                dmas.append(d)
            return dmas

        pending = issue_blk(0, 0)

        vaug_ref[...] = jnp.ones((SKV, HQ_PER * 128), jnp.bfloat16)

        row = lax.broadcasted_iota(jnp.int32, (SQ, SKV), 0)
        col = lax.broadcasted_iota(jnp.int32, (SQ, SKV), 1)
        bias_ref[...] = jnp.where(
            col // BLK <= row // BLK, 0.0, -1e9
        ).astype(jnp.float32)

        comm_wq[0] = wq_ref[...]
        comm_wo[0] = wo_ref[...]

        barrier_sem = pltpu.get_barrier_semaphore()
        for nbr in (left, right):
            pl.semaphore_signal(
                barrier_sem, inc=1,
                device_id=(nbr,), device_id_type=pl.DeviceIdType.MESH,
            )
        pl.semaphore_wait(barrier_sem, 2)

        def remote_copy(comm, src_slot, dst_slot, sem_idx, dev):
            return pltpu.make_async_remote_copy(
                src_ref=comm.at[src_slot], dst_ref=comm.at[dst_slot],
                send_sem=send_sems.at[sem_idx],
                recv_sem=recv_sems.at[sem_idx],
                device_id=(dev,),
                device_id_type=pl.DeviceIdType.MESH,
            )

        rdma_a_wq = remote_copy(comm_wq, 0, 1, 0, right)
        rdma_a_wo = remote_copy(comm_wo, 0, 1, 1, right)
        rdma_b_wq = remote_copy(comm_wq, 0, 2, 2, left)
        rdma_b_wo = remote_copy(comm_wo, 0, 2, 3, left)
        rdma_a_wq.start()
        rdma_b_wq.start()
        rdma_a_wo.start()
        rdma_b_wo.start()
        rdma_c_wq = remote_copy(comm_wq, 1, 3, 4, right)
        rdma_d_wo = remote_copy(comm_wo, 2, 3, 5, left)

        def compute_batch(slot, b, buf, wait_wo=None):
            wq = comm_wq[slot]
            q_ref[...] = (
                lax.dot_general(
                    x_ref[b], wq, (((1,), (0,)), ((), ())),
                    preferred_element_type=jnp.float32,
                )
                * 0.125
            ).astype(jnp.bfloat16)
            for h in range(HQ_PER):
                vaug_ref[:, h * 128:h * 128 + DH] = vblk[
                    buf, :, h * DH:(h + 1) * DH
                ]
            for h in range(HQ_PER):
                k_h = kblk[buf, :, h * DH:(h + 1) * DH]
                va_h = vaug_ref[:, h * 128:(h + 1) * 128]
                for lo, hi, nk in ((0, 128, 128), (128, 256, 256),
                                   (256, 384, 384), (384, SQ, SKV)):
                    scores = lax.dot_general(
                        q_ref[lo:hi, h * DH:(h + 1) * DH], k_h[:nk],
                        (((1,), (1,)), ((), ())),
                        preferred_element_type=jnp.float32,
                    )
                    e = jnp.exp(scores + bias_ref[lo:hi, :nk]).astype(
                        jnp.bfloat16
                    )
                    s_aug = lax.dot_general(
                        e, va_h[:nk], (((1,), (0,)), ((), ())),
                        preferred_element_type=jnp.float32,
                    )
                    ctx_ref[lo:hi, h * DH:(h + 1) * DH] = (
                        s_aug[:, :DH] / s_aug[:, DH:DH + 1]
                    ).astype(jnp.bfloat16)
            if wait_wo is not None:
                wait_wo.wait_recv()
            part = lax.dot_general(
                ctx_ref[...], comm_wo[slot], (((1,), (0,)), ((), ())),
                preferred_element_type=jnp.float32,
            )
            if slot == 0:
                out_ref[b] = part
            else:
                out_ref[b] = out_ref[b] + part

        for phase in range(N_PHASE):
            slot, b = divmod(phase, B_PER)
            buf = phase % 2
            wait_wo = None
            if phase == 2:
                rdma_a_wq.wait_recv()
                rdma_c_wq.start()
                wait_wo = rdma_a_wo
            if phase == 4:
                rdma_b_wq.wait_recv()
                rdma_b_wo.wait_recv()
                rdma_d_wo.start()
            if phase == 6:
                rdma_c_wq.wait_recv()
                rdma_d_wo.wait_recv()
            nxt = issue_blk(phase + 1, 1 - buf) if phase < N_PHASE - 1 else []
            for d in pending:
                d.wait()
            compute_batch(slot, b, buf, wait_wo)
            pending = nxt

        rdma_a_wq.wait_send()
        rdma_a_wo.wait_send()
        rdma_b_wq.wait_send()
        rdma_b_wo.wait_send()
        rdma_c_wq.wait_send()
        rdma_d_wo.wait_send()

    return pl.pallas_call(
        body,
        out_shape=jax.ShapeDtypeStruct((B_PER, SQ, DMODEL), jnp.float32),
        in_specs=[pl.BlockSpec(memory_space=pltpu.VMEM)] * 5,
        out_specs=pl.BlockSpec(memory_space=pltpu.VMEM),
        scratch_shapes=[
            pltpu.VMEM((N_DEV, DMODEL, HDIM), bf16),
            pltpu.VMEM((N_DEV, HDIM, DMODEL), bf16),
            pltpu.VMEM((SQ, HDIM), bf16),
            pltpu.VMEM((SQ, HDIM), bf16),
            pltpu.VMEM((SQ, SKV), jnp.float32),
            pltpu.VMEM((2, SKV, HDIM), bf16),
            pltpu.VMEM((2, SKV, HDIM), bf16),
            pltpu.VMEM((SKV, HQ_PER * 128), bf16),
            pltpu.SemaphoreType.DMA((6,)),
            pltpu.SemaphoreType.DMA((6,)),
            pltpu.SemaphoreType.DMA((2,)),
        ],
        compiler_params=_CompilerParams(collective_id=0),
    )(x_bf, K_ord, V_ord, Wq_bf, Wo_bf)


# device time: 60037 ns/iter; 1.1387x vs baseline; 1.1387x over previous
import jax
import jax.numpy as jnp
from jax import lax
from jax.experimental import pallas as pl
from jax.experimental.pallas import tpu as pltpu

N_DEV = 4
B_PER = 2
HQ_PER = 8
SQ = 512
SKV = 512
DH = 64
DMODEL = 768
HDIM = HQ_PER * DH
KVW = 32 * DH
BLK = 64
N_PHASE = N_DEV * B_PER

_CompilerParams = getattr(pltpu, "CompilerParams", None) or getattr(
    pltpu, "TPUCompilerParams"
)


def kernel(x, Wq, K_ext, V_ext, Wo):
    bf16 = jnp.bfloat16
    my = lax.axis_index("i")
    x_bf = x.astype(bf16)

    def prep(t):
        return lax.dynamic_slice_in_dim(
            t.reshape(8, SKV, KVW), my * B_PER, B_PER, axis=0
        ).astype(bf16)

    K_ord = prep(K_ext)
    V_ord = prep(V_ext)
    Wq_bf = Wq.astype(bf16)
    Wo_bf = Wo.astype(bf16)

    def body(
        x_ref, k_ref, v_ref, wq_ref, wo_ref, out_ref,
        comm_wq, comm_wo, q_ref, ctx_ref, bias_ref, kblk, vblk, vaug_ref,
        send_sems, recv_sems, blk_sems,
    ):
        my_i = lax.axis_index("i")
        left = lax.rem(my_i + N_DEV - 1, N_DEV)
        right = lax.rem(my_i + 1, N_DEV)
        origins = [my_i, left, right, lax.rem(my_i + 2, N_DEV)]

        def issue_blk(phase, buf):
            slot, b = divmod(phase, B_PER)
            off = origins[slot] * HDIM
            dmas = []
            for src, dst in ((k_ref, kblk), (v_ref, vblk)):
                d = pltpu.make_async_copy(
                    src.at[b, :, pl.ds(off, HDIM)],
                    dst.at[buf],
                    blk_sems.at[buf],
                )
                d.start()
                dmas.append(d)
            return dmas

        pending = issue_blk(0, 0)

        vaug_ref[...] = jnp.ones((SKV, HQ_PER * 128), jnp.bfloat16)

        row = lax.broadcasted_iota(jnp.int32, (SQ, SKV), 0)
        col = lax.broadcasted_iota(jnp.int32, (SQ, SKV), 1)
        bias_ref[...] = jnp.where(
            col // BLK <= row // BLK, 0.0, -1e9
        ).astype(jnp.float32)

        comm_wq[0] = wq_ref[...]
        comm_wo[0] = wo_ref[...]

        barrier_sem = pltpu.get_barrier_semaphore()
        for nbr in (left, right):
            pl.semaphore_signal(
                barrier_sem, inc=1,
                device_id=(nbr,), device_id_type=pl.DeviceIdType.MESH,
            )
        pl.semaphore_wait(barrier_sem, 2)

        def remote_copy(comm, src_slot, dst_slot, sem_idx, dev):
            return pltpu.make_async_remote_copy(
                src_ref=comm.at[src_slot], dst_ref=comm.at[dst_slot],
                send_sem=send_sems.at[sem_idx],
                recv_sem=recv_sems.at[sem_idx],
                device_id=(dev,),
                device_id_type=pl.DeviceIdType.MESH,
            )

        rdma_a_wq = remote_copy(comm_wq, 0, 1, 0, right)
        rdma_a_wo = remote_copy(comm_wo, 0, 1, 1, right)
        rdma_b_wq = remote_copy(comm_wq, 0, 2, 2, left)
        rdma_b_wo = remote_copy(comm_wo, 0, 2, 3, left)
        rdma_a_wq.start()
        rdma_b_wq.start()
        rdma_a_wo.start()
        rdma_b_wo.start()
        rdma_c_wq = remote_copy(comm_wq, 1, 3, 4, right)
        rdma_d_wo = remote_copy(comm_wo, 2, 3, 5, left)

        def compute_batch(slot, b, buf, wait_wo=None):
            wq = comm_wq[slot]
            q_ref[...] = (
                lax.dot_general(
                    x_ref[b], wq, (((1,), (0,)), ((), ())),
                    preferred_element_type=jnp.float32,
                )
                * 0.125
            ).astype(jnp.bfloat16)
            for h in range(HQ_PER):
                vaug_ref[:, h * 128:h * 128 + DH] = vblk[
                    buf, :, h * DH:(h + 1) * DH
                ]
            for h in range(HQ_PER):
                k_h = kblk[buf, :, h * DH:(h + 1) * DH]
                va_h = vaug_ref[:, h * 128:(h + 1) * 128]
                for lo, hi, nk in ((0, SQ // 2, SKV // 2),
                                   (SQ // 2, SQ, SKV)):
                    scores = lax.dot_general(
                        q_ref[lo:hi, h * DH:(h + 1) * DH], k_h[:nk],
                        (((1,), (1,)), ((), ())),
                        preferred_element_type=jnp.float32,
                    )
                    e = jnp.exp(scores + bias_ref[lo:hi, :nk]).astype(
                        jnp.bfloat16
                    )
                    s_aug = lax.dot_general(
                        e, va_h[:nk], (((1,), (0,)), ((), ())),
                        preferred_element_type=jnp.float32,
                    )
                    ctx_ref[lo:hi, h * DH:(h + 1) * DH] = (
                        s_aug[:, :DH] / s_aug[:, DH:DH + 1]
                    ).astype(jnp.bfloat16)
            if wait_wo is not None:
                wait_wo.wait_recv()
            part = lax.dot_general(
                ctx_ref[...], comm_wo[slot], (((1,), (0,)), ((), ())),
                preferred_element_type=jnp.float32,
            )
            if slot == 0:
                out_ref[b] = part
            else:
                out_ref[b] = out_ref[b] + part

        for phase in range(N_PHASE):
            slot, b = divmod(phase, B_PER)
            buf = phase % 2
            wait_wo = None
            if phase == 2:
                rdma_a_wq.wait_recv()
                rdma_c_wq.start()
                wait_wo = rdma_a_wo
            if phase == 4:
                rdma_b_wq.wait_recv()
                rdma_b_wo.wait_recv()
                rdma_d_wo.start()
            if phase == 6:
                rdma_c_wq.wait_recv()
                rdma_d_wo.wait_recv()
            nxt = issue_blk(phase + 1, 1 - buf) if phase < N_PHASE - 1 else []
            for d in pending:
                d.wait()
            compute_batch(slot, b, buf, wait_wo)
            pending = nxt

        rdma_a_wq.wait_send()
        rdma_a_wo.wait_send()
        rdma_b_wq.wait_send()
        rdma_b_wo.wait_send()
        rdma_c_wq.wait_send()
        rdma_d_wo.wait_send()

    return pl.pallas_call(
        body,
        out_shape=jax.ShapeDtypeStruct((B_PER, SQ, DMODEL), jnp.float32),
        in_specs=[pl.BlockSpec(memory_space=pltpu.VMEM)] * 5,
        out_specs=pl.BlockSpec(memory_space=pltpu.VMEM),
        scratch_shapes=[
            pltpu.VMEM((N_DEV, DMODEL, HDIM), bf16),
            pltpu.VMEM((N_DEV, HDIM, DMODEL), bf16),
            pltpu.VMEM((SQ, HDIM), bf16),
            pltpu.VMEM((SQ, HDIM), bf16),
            pltpu.VMEM((SQ, SKV), jnp.float32),
            pltpu.VMEM((2, SKV, HDIM), bf16),
            pltpu.VMEM((2, SKV, HDIM), bf16),
            pltpu.VMEM((SKV, HQ_PER * 128), bf16),
            pltpu.SemaphoreType.DMA((6,)),
            pltpu.SemaphoreType.DMA((6,)),
            pltpu.SemaphoreType.DMA((2,)),
        ],
        compiler_params=_CompilerParams(collective_id=0),
    )(x_bf, K_ord, V_ord, Wq_bf, Wo_bf)


# device time: 57506 ns/iter; 1.1888x vs baseline; 1.0440x over previous
import jax
import jax.numpy as jnp
from jax import lax
from jax.experimental import pallas as pl
from jax.experimental.pallas import tpu as pltpu

N_DEV = 4
B_PER = 2
HQ_PER = 8
SQ = 512
SKV = 512
DH = 64
DMODEL = 768
HDIM = HQ_PER * DH
KVW = 32 * DH
BLK = 64
N_PHASE = N_DEV * B_PER

_CompilerParams = getattr(pltpu, "CompilerParams", None) or getattr(
    pltpu, "TPUCompilerParams"
)


def kernel(x, Wq, K_ext, V_ext, Wo):
    bf16 = jnp.bfloat16
    my = lax.axis_index("i")
    x_bf = x.astype(bf16)

    def prep(t):
        return lax.dynamic_slice_in_dim(
            t.reshape(8, SKV, KVW), my * B_PER, B_PER, axis=0
        ).astype(bf16)

    K_ord = prep(K_ext)
    V_ord = prep(V_ext)
    Wq_bf = Wq.astype(bf16)
    Wo_bf = Wo.astype(bf16)

    def body(
        x_ref, k_ref, v_ref, wq_ref, wo_ref, out_ref,
        comm_wq, comm_wo, q_ref, ctx_ref, bias_ref, kblk, vblk, vaug_ref,
        send_sems, recv_sems, blk_sems,
    ):
        my_i = lax.axis_index("i")
        left = lax.rem(my_i + N_DEV - 1, N_DEV)
        right = lax.rem(my_i + 1, N_DEV)
        origins = [my_i, left, right, lax.rem(my_i + 2, N_DEV)]

        def issue_blk(phase, buf):
            slot, b = divmod(phase, B_PER)
            off = origins[slot] * HDIM
            dmas = []
            for src, dst in ((k_ref, kblk), (v_ref, vblk)):
                d = pltpu.make_async_copy(
                    src.at[b, :, pl.ds(off, HDIM)],
                    dst.at[buf],
                    blk_sems.at[buf],
                )
                d.start()
                dmas.append(d)
            return dmas

        pending = issue_blk(0, 0)

        vaug_ref[...] = jnp.ones((SKV, HQ_PER * 128), jnp.bfloat16)

        row = lax.broadcasted_iota(jnp.int32, (SQ, SKV), 0)
        col = lax.broadcasted_iota(jnp.int32, (SQ, SKV), 1)
        bias_ref[...] = jnp.where(
            col // BLK <= row // BLK, 0.0, -1e9
        ).astype(jnp.float32)

        comm_wq[0] = wq_ref[...]
        comm_wo[0] = wo_ref[...]

        barrier_sem = pltpu.get_barrier_semaphore()
        for nbr in (left, right):
            pl.semaphore_signal(
                barrier_sem, inc=1,
                device_id=(nbr,), device_id_type=pl.DeviceIdType.MESH,
            )
        pl.semaphore_wait(barrier_sem, 2)

        def remote_copy(comm, src_slot, dst_slot, sem_idx, dev):
            return pltpu.make_async_remote_copy(
                src_ref=comm.at[src_slot], dst_ref=comm.at[dst_slot],
                send_sem=send_sems.at[sem_idx],
                recv_sem=recv_sems.at[sem_idx],
                device_id=(dev,),
                device_id_type=pl.DeviceIdType.MESH,
            )

        rdma_a_wq = remote_copy(comm_wq, 0, 1, 0, right)
        rdma_a_wo = remote_copy(comm_wo, 0, 1, 1, right)
        rdma_b_wq = remote_copy(comm_wq, 0, 2, 2, left)
        rdma_b_wo = remote_copy(comm_wo, 0, 2, 3, left)
        rdma_a_wq.start()
        rdma_b_wq.start()
        rdma_a_wo.start()
        rdma_b_wo.start()
        rdma_c_wq = remote_copy(comm_wq, 1, 3, 4, right)
        rdma_d_wo = remote_copy(comm_wo, 2, 3, 5, left)

        def compute_batch(slot, b, buf):
            wq = comm_wq[slot]
            q_ref[...] = (
                lax.dot_general(
                    x_ref[b], wq, (((1,), (0,)), ((), ())),
                    preferred_element_type=jnp.float32,
                )
                * 0.125
            ).astype(jnp.bfloat16)
            for h in range(HQ_PER):
                vaug_ref[:, h * 128:h * 128 + DH] = vblk[
                    buf, :, h * DH:(h + 1) * DH
                ]
            for h in range(HQ_PER):
                k_h = kblk[buf, :, h * DH:(h + 1) * DH]
                va_h = vaug_ref[:, h * 128:(h + 1) * 128]
                for lo, hi, nk in ((0, SQ // 2, SKV // 2),
                                   (SQ // 2, SQ, SKV)):
                    scores = lax.dot_general(
                        q_ref[lo:hi, h * DH:(h + 1) * DH], k_h[:nk],
                        (((1,), (1,)), ((), ())),
                        preferred_element_type=jnp.float32,
                    )
                    e = jnp.exp(scores + bias_ref[lo:hi, :nk]).astype(
                        jnp.bfloat16
                    )
                    s_aug = lax.dot_general(
                        e, va_h[:nk], (((1,), (0,)), ((), ())),
                        preferred_element_type=jnp.float32,
                    )
                    col0 = slot * HDIM + h * DH
                    ctx_ref[b, lo:hi, col0:col0 + DH] = (
                        s_aug[:, :DH] / s_aug[:, DH:DH + 1]
                    ).astype(jnp.bfloat16)

        for phase in range(N_PHASE):
            slot, b = divmod(phase, B_PER)
            buf = phase % 2
            if phase == 2:
                rdma_a_wq.wait_recv()
                rdma_c_wq.start()
            if phase == 4:
                rdma_b_wq.wait_recv()
                rdma_b_wo.wait_recv()
                rdma_d_wo.start()
            if phase == 6:
                rdma_c_wq.wait_recv()
            nxt = issue_blk(phase + 1, 1 - buf) if phase < N_PHASE - 1 else []
            for d in pending:
                d.wait()
            compute_batch(slot, b, buf)
            pending = nxt

        rdma_a_wo.wait_recv()
        rdma_d_wo.wait_recv()
        for b in range(B_PER):
            acc = None
            for s in range(N_DEV):
                part = lax.dot_general(
                    ctx_ref[b, :, s * HDIM:(s + 1) * HDIM], comm_wo[s],
                    (((1,), (0,)), ((), ())),
                    preferred_element_type=jnp.float32,
                )
                acc = part if acc is None else acc + part
            out_ref[b] = acc

        rdma_a_wq.wait_send()
        rdma_a_wo.wait_send()
        rdma_b_wq.wait_send()
        rdma_b_wo.wait_send()
        rdma_c_wq.wait_send()
        rdma_d_wo.wait_send()

    return pl.pallas_call(
        body,
        out_shape=jax.ShapeDtypeStruct((B_PER, SQ, DMODEL), jnp.float32),
        in_specs=[pl.BlockSpec(memory_space=pltpu.VMEM)] * 5,
        out_specs=pl.BlockSpec(memory_space=pltpu.VMEM),
        scratch_shapes=[
            pltpu.VMEM((N_DEV, DMODEL, HDIM), bf16),
            pltpu.VMEM((N_DEV, HDIM, DMODEL), bf16),
            pltpu.VMEM((SQ, HDIM), bf16),
            pltpu.VMEM((B_PER, SQ, N_DEV * HDIM), bf16),
            pltpu.VMEM((SQ, SKV), jnp.float32),
            pltpu.VMEM((2, SKV, HDIM), bf16),
            pltpu.VMEM((2, SKV, HDIM), bf16),
            pltpu.VMEM((SKV, HQ_PER * 128), bf16),
            pltpu.SemaphoreType.DMA((6,)),
            pltpu.SemaphoreType.DMA((6,)),
            pltpu.SemaphoreType.DMA((2,)),
        ],
        compiler_params=_CompilerParams(collective_id=0),
    )(x_bf, K_ord, V_ord, Wq_bf, Wo_bf)
